# baseline (speedup 1.0000x reference)
import functools

import jax
import jax.numpy as jnp
from jax import lax
from jax.experimental import pallas as pl
from jax.experimental.pallas import tpu as pltpu
from jax.experimental.pallas import tpu_sc as plsc

_NC, _NS = 2, 16
_NW = _NC * _NS
_BLK = 2048


def _sc_gather(table, ids):
    batch = ids.shape[0]
    dim = table.shape[1]
    bpw = batch // _NW
    mesh = plsc.VectorSubcoreMesh(core_axis_name="c", subcore_axis_name="s")

    @functools.partial(
        pl.kernel,
        out_type=jax.ShapeDtypeStruct((batch, dim), jnp.float32),
        mesh=mesh,
        scratch_types=[
            pltpu.VMEM((bpw,), jnp.int32),
            pltpu.VMEM((bpw, dim), jnp.float32),
            pltpu.SemaphoreType.DMA,
        ],
        compiler_params=pltpu.CompilerParams(use_tc_tiling_on_sc=False),
    )
    def k(table_hbm, idx_hbm, out_hbm, idx_v, rows_v, sem):
        wid = lax.axis_index("s") * _NC + lax.axis_index("c")
        base = wid * bpw
        pltpu.sync_copy(idx_hbm.at[pl.ds(base, bpw)], idx_v)
        pltpu.async_copy(table_hbm.at[idx_v], rows_v, sem).wait()
        pltpu.sync_copy(rows_v, out_hbm.at[pl.ds(base, bpw)])

    return k(table, ids)


def _mlp_body(emb_ref, sv_ref, Wp_ref, bp_ref, W1_ref, b1_ref, W2_ref,
              b2_ref, out_ref):
    ed = emb_ref.shape[1]
    proj = jnp.dot(sv_ref[...], Wp_ref[...],
                   preferred_element_type=jnp.float32) + bp_ref[...]
    w1 = W1_ref[...]
    h = jnp.dot(emb_ref[...], w1[:ed], preferred_element_type=jnp.float32)
    h = h + jnp.dot(proj, w1[ed:], preferred_element_type=jnp.float32)
    h = jnp.maximum(h + b1_ref[...], 0.0)
    out_ref[...] = jnp.dot(h, W2_ref[...],
                           preferred_element_type=jnp.float32) + b2_ref[...]


def _mlp(emb, sv, Wp, bp, W1, b1, W2, b2):
    batch, ed = emb.shape
    sd = sv.shape[1]
    hd = W1.shape[1]
    blk = min(_BLK, batch)
    full = lambda *shape: pl.BlockSpec(shape, lambda i: (0,) * len(shape))
    return pl.pallas_call(
        _mlp_body,
        grid=(batch // blk,),
        in_specs=[
            pl.BlockSpec((blk, ed), lambda i: (i, 0)),
            pl.BlockSpec((blk, sd), lambda i: (i, 0)),
            full(sd, ed),
            full(1, ed),
            full(2 * ed, hd),
            full(1, hd),
            full(hd, ed),
            full(1, ed),
        ],
        out_specs=pl.BlockSpec((blk, ed), lambda i: (i, 0)),
        out_shape=jax.ShapeDtypeStruct((batch, ed), jnp.float32),
    )(emb, sv, Wp, bp.reshape(1, -1), W1, b1.reshape(1, -1), W2,
      b2.reshape(1, -1))


def kernel(movie_ids, semantic_vectors, table, Wp, bp, W1, b1, W2, b2):
    emb = _sc_gather(table, movie_ids.astype(jnp.int32))
    return _mlp(emb, semantic_vectors, Wp, bp, W1, b1, W2, b2)


# speedup vs baseline: 1.6840x; 1.6840x over previous
import functools

import jax
import jax.numpy as jnp
from jax import lax
from jax.experimental import pallas as pl
from jax.experimental.pallas import tpu as pltpu
from jax.experimental.pallas import tpu_sc as plsc

_NC, _NS = 2, 16
_NW = _NC * _NS
_BLK = 2048


def _sc_gather(table, ids):
    batch = ids.shape[0]
    dim = table.shape[1]
    bpw = batch // _NW
    mesh = plsc.VectorSubcoreMesh(core_axis_name="c", subcore_axis_name="s")

    @functools.partial(
        pl.kernel,
        out_type=jax.ShapeDtypeStruct((batch, dim), jnp.float32),
        mesh=mesh,
        scratch_types=[
            pltpu.VMEM((bpw,), jnp.int32),
            pltpu.VMEM((bpw, dim), jnp.float32),
            pltpu.SemaphoreType.DMA,
        ],
    )
    def k(table_hbm, idx_hbm, out_hbm, idx_v, rows_v, sem):
        wid = lax.axis_index("s") * _NC + lax.axis_index("c")
        base = wid * bpw
        pltpu.sync_copy(idx_hbm.at[pl.ds(base, bpw)], idx_v)

        def fire(g, carry):
            vec = idx_v[pl.ds(g * 16, 16)]
            for j in range(16):
                pltpu.make_async_copy(
                    table_hbm.at[vec[j]], rows_v.at[g * 16 + j], sem).start()
            return carry

        lax.fori_loop(0, bpw // 16, fire, 0)

        def drain(r, carry):
            pltpu.make_async_copy(
                table_hbm.at[0], rows_v.at[r], sem).wait()
            return carry

        lax.fori_loop(0, bpw, drain, 0)
        pltpu.sync_copy(rows_v, out_hbm.at[pl.ds(base, bpw)])

    return k(table, ids)


def _mlp_body(emb_ref, sv_ref, Wp_ref, bp_ref, W1_ref, b1_ref, W2_ref,
              b2_ref, out_ref):
    ed = emb_ref.shape[1]
    proj = jnp.dot(sv_ref[...], Wp_ref[...],
                   preferred_element_type=jnp.float32) + bp_ref[...]
    w1 = W1_ref[...]
    h = jnp.dot(emb_ref[...], w1[:ed], preferred_element_type=jnp.float32)
    h = h + jnp.dot(proj, w1[ed:], preferred_element_type=jnp.float32)
    h = jnp.maximum(h + b1_ref[...], 0.0)
    out_ref[...] = jnp.dot(h, W2_ref[...],
                           preferred_element_type=jnp.float32) + b2_ref[...]


def _mlp(emb, sv, Wp, bp, W1, b1, W2, b2):
    batch, ed = emb.shape
    sd = sv.shape[1]
    hd = W1.shape[1]
    blk = min(_BLK, batch)
    full = lambda *shape: pl.BlockSpec(shape, lambda i: (0,) * len(shape))
    return pl.pallas_call(
        _mlp_body,
        grid=(batch // blk,),
        in_specs=[
            pl.BlockSpec((blk, ed), lambda i: (i, 0)),
            pl.BlockSpec((blk, sd), lambda i: (i, 0)),
            full(sd, ed),
            full(1, ed),
            full(2 * ed, hd),
            full(1, hd),
            full(hd, ed),
            full(1, ed),
        ],
        out_specs=pl.BlockSpec((blk, ed), lambda i: (i, 0)),
        out_shape=jax.ShapeDtypeStruct((batch, ed), jnp.float32),
    )(emb, sv, Wp, bp.reshape(1, -1), W1, b1.reshape(1, -1), W2,
      b2.reshape(1, -1))


def kernel(movie_ids, semantic_vectors, table, Wp, bp, W1, b1, W2, b2):
    emb = _sc_gather(table, movie_ids.astype(jnp.int32))
    return _mlp(emb, semantic_vectors, Wp, bp, W1, b1, W2, b2)
